# baseline (speedup 1.0000x reference)
import functools

import jax
import jax.numpy as jnp
from jax import lax
from jax.experimental import pallas as pl
from jax.experimental.pallas import tpu as pltpu
from jax.experimental.pallas import tpu_sc as plsc

N = 10000
NPAD = 10240
D = 128
E = 320000
NC, NS = 2, 16
NW = NC * NS
EC = 100
SUP = 50
NSUP = 2
CH = SUP * NSUP
ROWS_PT = NPAD // NS
RB = 1280
GRID = NPAD // RB

_mesh = plsc.VectorSubcoreMesh(core_axis_name="c", subcore_axis_name="s")


def _deg_body(dst_hbm, deg_out, dst_v, ones_v, zero_v, deg_sh, sem_s):
    c = lax.axis_index("c")
    s = lax.axis_index("s")
    wid = s * NC + c
    row0 = s * ROWS_PT

    one = jnp.full((16,), 1.0, jnp.float32)
    zero = jnp.zeros((16,), jnp.float32)
    for i in range(EC):
        for j in range(D // 16):
            ones_v[i, pl.ds(j * 16, 16)] = one
    for i in range(16):
        for j in range(D // 16):
            zero_v[i, pl.ds(j * 16, 16)] = zero

    for u in range(NSUP):
        pltpu.sync_copy(dst_hbm.at[wid * NSUP + u], dst_v.at[pl.ds(u * SUP, SUP)])

    for k in range(ROWS_PT // 16):
        pltpu.sync_copy(zero_v, deg_sh.at[pl.ds(row0 + k * 16, 16)])
    plsc.subcore_barrier()

    def fire(t, carry):
        pltpu.async_copy(ones_v, deg_sh.at[dst_v.at[t]], sem_s, add=True)
        return carry

    lax.fori_loop(0, CH, fire, 0)

    def drain(t, carry):
        pltpu.make_async_copy(ones_v, deg_sh.at[dst_v.at[0]], sem_s).wait()
        return carry

    lax.fori_loop(0, CH, drain, 0)
    plsc.subcore_barrier()

    pltpu.sync_copy(deg_sh.at[pl.ds(row0, ROWS_PT)],
                    deg_out.at[pl.ds(c * NPAD + row0, ROWS_PT)])


_deg_call = functools.partial(
    pl.kernel,
    out_type=jax.ShapeDtypeStruct((NC * NPAD, D), jnp.float32),
    mesh=_mesh,
    scratch_types=[
        pltpu.VMEM((CH, EC), jnp.int32),
        pltpu.VMEM((EC, D), jnp.float32),
        pltpu.VMEM((16, D), jnp.float32),
        pltpu.VMEM_SHARED((NPAD, D), jnp.float32),
        pltpu.SemaphoreType.DMA,
    ],
)(_deg_body)


def _scat_body(g_hbm, src_hbm, dst_hbm, acc_out, src_v, dst_v, rows_a, rows_b,
               acc_sh, sem_g, sem_s):
    c = lax.axis_index("c")
    s = lax.axis_index("s")
    wid = s * NC + c
    row0 = s * ROWS_PT

    pltpu.sync_copy(src_hbm.at[wid * NSUP], src_v)
    pltpu.sync_copy(dst_hbm.at[wid * NSUP], dst_v)
    pltpu.async_copy(g_hbm.at[src_v.at[0]], rows_a, sem_g)

    zero = jnp.zeros((16,), jnp.float32)
    for i in range(16):
        for j in range(D // 16):
            rows_b[i, pl.ds(j * 16, 16)] = zero
    for k in range(ROWS_PT // 16):
        pltpu.sync_copy(rows_b.at[pl.ds(0, 16)],
                        acc_sh.at[pl.ds(row0 + k * 16, 16)])
    plsc.subcore_barrier()

    def _drain_one():
        pltpu.make_async_copy(rows_b, acc_sh.at[dst_v.at[0]], sem_s).wait()

    def _slab_pipe():
        def body(i, carry):
            t0 = 2 * i

            pltpu.make_async_copy(g_hbm.at[src_v.at[t0]], rows_a, sem_g).wait()

            @pl.when(t0 >= 1)
            def _():
                _drain_one()

            pltpu.async_copy(g_hbm.at[src_v.at[t0 + 1]], rows_b, sem_g)
            pltpu.async_copy(rows_a, acc_sh.at[dst_v.at[t0]], sem_s, add=True)

            pltpu.make_async_copy(g_hbm.at[src_v.at[t0 + 1]], rows_b,
                                  sem_g).wait()
            _drain_one()

            @pl.when(t0 + 2 < SUP)
            def _():
                pltpu.async_copy(g_hbm.at[src_v.at[t0 + 2]], rows_a, sem_g)

            pltpu.async_copy(rows_b, acc_sh.at[dst_v.at[t0 + 1]], sem_s,
                             add=True)
            return carry

        lax.fori_loop(0, SUP // 2, body, 0)
        _drain_one()

    _slab_pipe()
    pltpu.sync_copy(src_hbm.at[wid * NSUP + 1], src_v)
    pltpu.sync_copy(dst_hbm.at[wid * NSUP + 1], dst_v)
    pltpu.async_copy(g_hbm.at[src_v.at[0]], rows_a, sem_g)
    _slab_pipe()
    plsc.subcore_barrier()

    pltpu.sync_copy(acc_sh.at[pl.ds(row0, ROWS_PT)],
                    acc_out.at[pl.ds(c * NPAD + row0, ROWS_PT)])


_scat_call = functools.partial(
    pl.kernel,
    out_type=jax.ShapeDtypeStruct((NC * NPAD, D), jnp.float32),
    mesh=_mesh,
    scratch_types=[
        pltpu.VMEM((SUP, EC), jnp.int32),
        pltpu.VMEM((SUP, EC), jnp.int32),
        pltpu.VMEM((EC, D), jnp.float32),
        pltpu.VMEM((EC, D), jnp.float32),
        pltpu.VMEM_SHARED((NPAD, D), jnp.float32),
        pltpu.SemaphoreType.DMA,
        pltpu.SemaphoreType.DMA,
    ],
)(_scat_body)


def _tc1_body(x_ref, w_ref, d0_ref, d1_ref, g_ref, dinv_ref):
    deg = jnp.maximum(d0_ref[:, 0:1] + d1_ref[:, 0:1] + 1.0, 1.0)
    dinv = lax.rsqrt(deg)
    h = jnp.dot(x_ref[...], w_ref[...], preferred_element_type=jnp.float32)
    g_ref[...] = h * dinv
    dinv_ref[...] = jnp.broadcast_to(dinv, dinv_ref.shape)


def _tc1(x, w1, d0, d1):
    return pl.pallas_call(
        _tc1_body,
        grid=(GRID,),
        in_specs=[
            pl.BlockSpec((RB, D), lambda i: (i, 0)),
            pl.BlockSpec((D, D), lambda i: (0, 0)),
            pl.BlockSpec((RB, D), lambda i: (i, 0)),
            pl.BlockSpec((RB, D), lambda i: (i, 0)),
        ],
        out_specs=[
            pl.BlockSpec((RB, D), lambda i: (i, 0)),
            pl.BlockSpec((RB, 16), lambda i: (i, 0)),
        ],
        out_shape=[
            jax.ShapeDtypeStruct((NPAD, D), jnp.float32),
            jax.ShapeDtypeStruct((NPAD, 16), jnp.float32),
        ],
    )(x, w1, d0, d1)


def _tc2_body(a0_ref, a1_ref, g_ref, dinv_ref, b_ref, w_ref, g2_ref):
    i = pl.program_id(0)
    dinv = dinv_ref[:, 0:1]
    z = dinv * (a0_ref[...] + a1_ref[...] + g_ref[...]) + b_ref[...]
    z = jnp.maximum(z, 0.0)
    rows = i * RB + lax.broadcasted_iota(jnp.int32, (RB, 1), 0)
    z = jnp.where(rows < N, z, 0.0)
    h2 = jnp.dot(z, w_ref[...], preferred_element_type=jnp.float32)
    g2_ref[...] = h2 * dinv


def _tc2(a0, a1, g, dinv, b1, w2):
    return pl.pallas_call(
        _tc2_body,
        grid=(GRID,),
        in_specs=[
            pl.BlockSpec((RB, D), lambda i: (i, 0)),
            pl.BlockSpec((RB, D), lambda i: (i, 0)),
            pl.BlockSpec((RB, D), lambda i: (i, 0)),
            pl.BlockSpec((RB, 16), lambda i: (i, 0)),
            pl.BlockSpec((1, D), lambda i: (0, 0)),
            pl.BlockSpec((D, D), lambda i: (0, 0)),
        ],
        out_specs=pl.BlockSpec((RB, D), lambda i: (i, 0)),
        out_shape=jax.ShapeDtypeStruct((NPAD, D), jnp.float32),
    )(a0, a1, g, dinv, b1, w2)


def _tc3_body(a0_ref, a1_ref, g_ref, dinv_ref, b_ref, out_ref):
    dinv = dinv_ref[:, 0:1]
    z = dinv * (a0_ref[...] + a1_ref[...] + g_ref[...]) + b_ref[...]
    out_ref[...] = jnp.maximum(z, 0.0)


def _tc3(a0, a1, g, dinv, b2):
    return pl.pallas_call(
        _tc3_body,
        grid=(GRID,),
        in_specs=[
            pl.BlockSpec((RB, D), lambda i: (i, 0)),
            pl.BlockSpec((RB, D), lambda i: (i, 0)),
            pl.BlockSpec((RB, D), lambda i: (i, 0)),
            pl.BlockSpec((RB, 16), lambda i: (i, 0)),
            pl.BlockSpec((1, D), lambda i: (0, 0)),
        ],
        out_specs=pl.BlockSpec((RB, D), lambda i: (i, 0)),
        out_shape=jax.ShapeDtypeStruct((NPAD, D), jnp.float32),
    )(a0, a1, g, dinv, b2)


def kernel(x, edge_index, W1, b1, W2, b2):
    src = edge_index[0].astype(jnp.int32)
    dst = edge_index[1].astype(jnp.int32)
    src_p = src.reshape(NW * NSUP, SUP, EC)
    dst_p = dst.reshape(NW * NSUP, SUP, EC)

    x_p = jnp.pad(x, ((0, NPAD - N), (0, 0)))
    b1r = b1.reshape(1, D)
    b2r = b2.reshape(1, D)

    deg = _deg_call(dst_p)
    d0 = deg[:NPAD]
    d1 = deg[NPAD:]

    g1, dinv = _tc1(x_p, W1, d0, d1)
    acc1 = _scat_call(g1, src_p, dst_p)
    g2 = _tc2(acc1[:NPAD], acc1[NPAD:], g1, dinv, b1r, W2)
    acc2 = _scat_call(g2, src_p, dst_p)
    out = _tc3(acc2[:NPAD], acc2[NPAD:], g2, dinv, b2r)
    return out[:N]


# speedup vs baseline: 1.0620x; 1.0620x over previous
import functools

import jax
import jax.numpy as jnp
from jax import lax
from jax.experimental import pallas as pl
from jax.experimental.pallas import tpu as pltpu
from jax.experimental.pallas import tpu_sc as plsc

N = 10000
NPAD = 10240
D = 128
E = 320000
NC, NS = 2, 16
NW = NC * NS
EC = 100
SUP = 50
NSUP = 2
CH = SUP * NSUP
ROWS_PT = NPAD // NS
RB = 1280
GRID = NPAD // RB

_mesh = plsc.VectorSubcoreMesh(core_axis_name="c", subcore_axis_name="s")


def _deg_body(dst_hbm, deg_out0, deg_out1, dst_v, ones_v, zero_v, deg_sh,
              sem_s):
    c = lax.axis_index("c")
    s = lax.axis_index("s")
    wid = s * NC + c
    row0 = s * ROWS_PT

    one = jnp.full((16,), 1.0, jnp.float32)
    zero = jnp.zeros((16,), jnp.float32)
    for i in range(EC):
        for j in range(D // 16):
            ones_v[i, pl.ds(j * 16, 16)] = one
    for i in range(16):
        for j in range(D // 16):
            zero_v[i, pl.ds(j * 16, 16)] = zero

    for u in range(NSUP):
        pltpu.sync_copy(dst_hbm.at[wid * NSUP + u], dst_v.at[pl.ds(u * SUP, SUP)])

    for k in range(ROWS_PT // 16):
        pltpu.sync_copy(zero_v, deg_sh.at[pl.ds(row0 + k * 16, 16)])
    plsc.subcore_barrier()

    def fire(t, carry):
        pltpu.async_copy(ones_v, deg_sh.at[dst_v.at[t]], sem_s, add=True)
        return carry

    lax.fori_loop(0, CH, fire, 0)

    def drain(t, carry):
        pltpu.make_async_copy(ones_v, deg_sh.at[dst_v.at[0]], sem_s).wait()
        return carry

    lax.fori_loop(0, CH, drain, 0)
    plsc.subcore_barrier()

    @pl.when(c == 0)
    def _():
        pltpu.sync_copy(deg_sh.at[pl.ds(row0, ROWS_PT)],
                        deg_out0.at[pl.ds(row0, ROWS_PT)])

    @pl.when(c == 1)
    def _():
        pltpu.sync_copy(deg_sh.at[pl.ds(row0, ROWS_PT)],
                        deg_out1.at[pl.ds(row0, ROWS_PT)])


_deg_call = functools.partial(
    pl.kernel,
    out_type=[jax.ShapeDtypeStruct((NPAD, D), jnp.float32),
              jax.ShapeDtypeStruct((NPAD, D), jnp.float32)],
    mesh=_mesh,
    scratch_types=[
        pltpu.VMEM((CH, EC), jnp.int32),
        pltpu.VMEM((EC, D), jnp.float32),
        pltpu.VMEM((16, D), jnp.float32),
        pltpu.VMEM_SHARED((NPAD, D), jnp.float32),
        pltpu.SemaphoreType.DMA,
    ],
)(_deg_body)


def _scat_body(g_hbm, src_hbm, dst_hbm, acc_out0, acc_out1, src_v, dst_v,
               rows_a, rows_b, acc_sh, sem_g, sem_s):
    c = lax.axis_index("c")
    s = lax.axis_index("s")
    wid = s * NC + c
    row0 = s * ROWS_PT

    pltpu.sync_copy(src_hbm.at[wid * NSUP], src_v)
    pltpu.sync_copy(dst_hbm.at[wid * NSUP], dst_v)
    pltpu.async_copy(g_hbm.at[src_v.at[0]], rows_a, sem_g)

    zero = jnp.zeros((16,), jnp.float32)
    for i in range(16):
        for j in range(D // 16):
            rows_b[i, pl.ds(j * 16, 16)] = zero
    for k in range(ROWS_PT // 16):
        pltpu.sync_copy(rows_b.at[pl.ds(0, 16)],
                        acc_sh.at[pl.ds(row0 + k * 16, 16)])
    plsc.subcore_barrier()

    def _drain_one():
        pltpu.make_async_copy(rows_b, acc_sh.at[dst_v.at[0]], sem_s).wait()

    def _slab_pipe():
        def body(i, carry):
            t0 = 2 * i

            pltpu.make_async_copy(g_hbm.at[src_v.at[t0]], rows_a, sem_g).wait()

            @pl.when(t0 >= 1)
            def _():
                _drain_one()

            pltpu.async_copy(g_hbm.at[src_v.at[t0 + 1]], rows_b, sem_g)
            pltpu.async_copy(rows_a, acc_sh.at[dst_v.at[t0]], sem_s, add=True)

            pltpu.make_async_copy(g_hbm.at[src_v.at[t0 + 1]], rows_b,
                                  sem_g).wait()
            _drain_one()

            @pl.when(t0 + 2 < SUP)
            def _():
                pltpu.async_copy(g_hbm.at[src_v.at[t0 + 2]], rows_a, sem_g)

            pltpu.async_copy(rows_b, acc_sh.at[dst_v.at[t0 + 1]], sem_s,
                             add=True)
            return carry

        lax.fori_loop(0, SUP // 2, body, 0)
        _drain_one()

    _slab_pipe()
    pltpu.sync_copy(src_hbm.at[wid * NSUP + 1], src_v)
    pltpu.sync_copy(dst_hbm.at[wid * NSUP + 1], dst_v)
    pltpu.async_copy(g_hbm.at[src_v.at[0]], rows_a, sem_g)
    _slab_pipe()
    plsc.subcore_barrier()

    @pl.when(c == 0)
    def _():
        pltpu.sync_copy(acc_sh.at[pl.ds(row0, ROWS_PT)],
                        acc_out0.at[pl.ds(row0, ROWS_PT)])

    @pl.when(c == 1)
    def _():
        pltpu.sync_copy(acc_sh.at[pl.ds(row0, ROWS_PT)],
                        acc_out1.at[pl.ds(row0, ROWS_PT)])


_scat_call = functools.partial(
    pl.kernel,
    out_type=[jax.ShapeDtypeStruct((NPAD, D), jnp.float32),
              jax.ShapeDtypeStruct((NPAD, D), jnp.float32)],
    mesh=_mesh,
    scratch_types=[
        pltpu.VMEM((SUP, EC), jnp.int32),
        pltpu.VMEM((SUP, EC), jnp.int32),
        pltpu.VMEM((EC, D), jnp.float32),
        pltpu.VMEM((EC, D), jnp.float32),
        pltpu.VMEM_SHARED((NPAD, D), jnp.float32),
        pltpu.SemaphoreType.DMA,
        pltpu.SemaphoreType.DMA,
    ],
)(_scat_body)


def _tc1_body(x_ref, w_ref, d0_ref, d1_ref, g_ref, dinv_ref):
    deg = jnp.maximum(d0_ref[:, 0:1] + d1_ref[:, 0:1] + 1.0, 1.0)
    dinv = lax.rsqrt(deg)
    h = jnp.dot(x_ref[...], w_ref[...], preferred_element_type=jnp.float32)
    g_ref[...] = h * dinv
    dinv_ref[...] = jnp.broadcast_to(dinv, dinv_ref.shape)


def _tc1(x, w1, d0, d1):
    return pl.pallas_call(
        _tc1_body,
        grid=(GRID,),
        in_specs=[
            pl.BlockSpec((RB, D), lambda i: (i, 0)),
            pl.BlockSpec((D, D), lambda i: (0, 0)),
            pl.BlockSpec((RB, D), lambda i: (i, 0)),
            pl.BlockSpec((RB, D), lambda i: (i, 0)),
        ],
        out_specs=[
            pl.BlockSpec((RB, D), lambda i: (i, 0)),
            pl.BlockSpec((RB, 16), lambda i: (i, 0)),
        ],
        out_shape=[
            jax.ShapeDtypeStruct((NPAD, D), jnp.float32),
            jax.ShapeDtypeStruct((NPAD, 16), jnp.float32),
        ],
    )(x, w1, d0, d1)


def _tc2_body(a0_ref, a1_ref, g_ref, dinv_ref, b_ref, w_ref, g2_ref):
    i = pl.program_id(0)
    dinv = dinv_ref[:, 0:1]
    z = dinv * (a0_ref[...] + a1_ref[...] + g_ref[...]) + b_ref[...]
    z = jnp.maximum(z, 0.0)
    rows = i * RB + lax.broadcasted_iota(jnp.int32, (RB, 1), 0)
    z = jnp.where(rows < N, z, 0.0)
    h2 = jnp.dot(z, w_ref[...], preferred_element_type=jnp.float32)
    g2_ref[...] = h2 * dinv


def _tc3_n_body(a0_ref, a1_ref, g_ref, dinv_ref, b_ref, out_ref):
    dinv = dinv_ref[:, 0:1]
    z = dinv * (a0_ref[...] + a1_ref[...] + g_ref[...]) + b_ref[...]
    out_ref[...] = jnp.maximum(z, 0.0)


def _tc3_n(a0, a1, g, dinv, b2):
    nb = 2000
    return pl.pallas_call(
        _tc3_n_body,
        grid=(N // nb,),
        in_specs=[
            pl.BlockSpec((nb, D), lambda i: (i, 0)),
            pl.BlockSpec((nb, D), lambda i: (i, 0)),
            pl.BlockSpec((nb, D), lambda i: (i, 0)),
            pl.BlockSpec((nb, 16), lambda i: (i, 0)),
            pl.BlockSpec((1, D), lambda i: (0, 0)),
        ],
        out_specs=pl.BlockSpec((nb, D), lambda i: (i, 0)),
        out_shape=jax.ShapeDtypeStruct((N, D), jnp.float32),
    )(a0, a1, g, dinv, b2)


def _tc2(a0, a1, g, dinv, b1, w2):
    return pl.pallas_call(
        _tc2_body,
        grid=(GRID,),
        in_specs=[
            pl.BlockSpec((RB, D), lambda i: (i, 0)),
            pl.BlockSpec((RB, D), lambda i: (i, 0)),
            pl.BlockSpec((RB, D), lambda i: (i, 0)),
            pl.BlockSpec((RB, 16), lambda i: (i, 0)),
            pl.BlockSpec((1, D), lambda i: (0, 0)),
            pl.BlockSpec((D, D), lambda i: (0, 0)),
        ],
        out_specs=pl.BlockSpec((RB, D), lambda i: (i, 0)),
        out_shape=jax.ShapeDtypeStruct((NPAD, D), jnp.float32),
    )(a0, a1, g, dinv, b1, w2)


def kernel(x, edge_index, W1, b1, W2, b2):
    src = edge_index[0].astype(jnp.int32)
    dst = edge_index[1].astype(jnp.int32)
    src_p = src.reshape(NW * NSUP, SUP, EC)
    dst_p = dst.reshape(NW * NSUP, SUP, EC)

    x_p = jnp.pad(x, ((0, NPAD - N), (0, 0)))
    b1r = b1.reshape(1, D)
    b2r = b2.reshape(1, D)

    d0, d1 = _deg_call(dst_p)
    g1, dinv = _tc1(x_p, W1, d0, d1)
    a0, a1 = _scat_call(g1, src_p, dst_p)
    g2 = _tc2(a0, a1, g1, dinv, b1r, W2)
    a0, a1 = _scat_call(g2, src_p, dst_p)
    return _tc3_n(a0, a1, g2, dinv, b2r)


# speedup vs baseline: 1.0667x; 1.0044x over previous
import functools

import jax
import jax.numpy as jnp
from jax import lax
from jax.experimental import pallas as pl
from jax.experimental.pallas import tpu as pltpu
from jax.experimental.pallas import tpu_sc as plsc

N = 10000
NPAD = 10240
D = 128
E = 320000
NC, NS = 2, 16
NW = NC * NS
EC = 100
SUP = 50
NSUP = 2
CH = SUP * NSUP
ROWS_PT = NPAD // NS
RB = 1280
GRID = NPAD // RB

_mesh = plsc.VectorSubcoreMesh(core_axis_name="c", subcore_axis_name="s")


def _deg_body(dst_hbm, deg_out0, deg_out1, dst_v, ones_v, zero_v, deg_sh,
              sem_s):
    c = lax.axis_index("c")
    s = lax.axis_index("s")
    wid = s * NC + c
    row0 = s * ROWS_PT

    one = jnp.full((16,), 1.0, jnp.float32)
    zero = jnp.zeros((16,), jnp.float32)
    for i in range(EC):
        for j in range(D // 16):
            ones_v[i, pl.ds(j * 16, 16)] = one
    for i in range(16):
        for j in range(D // 16):
            zero_v[i, pl.ds(j * 16, 16)] = zero

    for u in range(NSUP):
        pltpu.sync_copy(dst_hbm.at[wid * NSUP + u], dst_v.at[pl.ds(u * SUP, SUP)])

    for k in range(ROWS_PT // 16):
        pltpu.sync_copy(zero_v, deg_sh.at[pl.ds(row0 + k * 16, 16)])
    plsc.subcore_barrier()

    def fire(t, carry):
        pltpu.async_copy(ones_v, deg_sh.at[dst_v.at[t]], sem_s, add=True)
        return carry

    lax.fori_loop(0, CH, fire, 0)

    def drain(t, carry):
        pltpu.make_async_copy(ones_v, deg_sh.at[dst_v.at[0]], sem_s).wait()
        return carry

    lax.fori_loop(0, CH, drain, 0)
    plsc.subcore_barrier()

    @pl.when(c == 0)
    def _():
        pltpu.sync_copy(deg_sh.at[pl.ds(row0, ROWS_PT)],
                        deg_out0.at[pl.ds(row0, ROWS_PT)])

    @pl.when(c == 1)
    def _():
        pltpu.sync_copy(deg_sh.at[pl.ds(row0, ROWS_PT)],
                        deg_out1.at[pl.ds(row0, ROWS_PT)])


_deg_call = functools.partial(
    pl.kernel,
    out_type=[jax.ShapeDtypeStruct((NPAD, D), jnp.float32),
              jax.ShapeDtypeStruct((NPAD, D), jnp.float32)],
    mesh=_mesh,
    scratch_types=[
        pltpu.VMEM((CH, EC), jnp.int32),
        pltpu.VMEM((EC, D), jnp.float32),
        pltpu.VMEM((16, D), jnp.float32),
        pltpu.VMEM_SHARED((NPAD, D), jnp.float32),
        pltpu.SemaphoreType.DMA,
    ],
)(_deg_body)


def _scat_body(g_hbm, src_hbm, dst_hbm, acc_out0, acc_out1, src_v, dst_v,
               rows_a, rows_b, acc_sh, sem_g, sem_s):
    c = lax.axis_index("c")
    s = lax.axis_index("s")
    wid = s * NC + c
    row0 = s * ROWS_PT

    pltpu.sync_copy(src_hbm.at[wid * NSUP], src_v)
    pltpu.sync_copy(dst_hbm.at[wid * NSUP], dst_v)
    pltpu.async_copy(g_hbm.at[src_v.at[0]], rows_a, sem_g)

    zero = jnp.zeros((16,), jnp.float32)
    for i in range(16):
        for j in range(D // 16):
            rows_b[i, pl.ds(j * 16, 16)] = zero
    for k in range(ROWS_PT // 16):
        pltpu.sync_copy(rows_b.at[pl.ds(0, 16)],
                        acc_sh.at[pl.ds(row0 + k * 16, 16)])
    plsc.subcore_barrier()

    def _slab_pipe():
        def body(i, carry):
            t0 = 2 * i

            pltpu.make_async_copy(g_hbm.at[src_v.at[t0]], rows_a, sem_g).wait()
            pltpu.async_copy(g_hbm.at[src_v.at[t0 + 1]], rows_b, sem_g)
            pltpu.sync_copy(rows_a, acc_sh.at[dst_v.at[t0]], add=True)

            pltpu.make_async_copy(g_hbm.at[src_v.at[t0 + 1]], rows_b,
                                  sem_g).wait()

            @pl.when(t0 + 2 < SUP)
            def _():
                pltpu.async_copy(g_hbm.at[src_v.at[t0 + 2]], rows_a, sem_g)

            pltpu.sync_copy(rows_b, acc_sh.at[dst_v.at[t0 + 1]], add=True)
            return carry

        lax.fori_loop(0, SUP // 2, body, 0)

    _slab_pipe()
    pltpu.sync_copy(src_hbm.at[wid * NSUP + 1], src_v)
    pltpu.sync_copy(dst_hbm.at[wid * NSUP + 1], dst_v)
    pltpu.async_copy(g_hbm.at[src_v.at[0]], rows_a, sem_g)
    _slab_pipe()
    plsc.subcore_barrier()

    @pl.when(c == 0)
    def _():
        pltpu.sync_copy(acc_sh.at[pl.ds(row0, ROWS_PT)],
                        acc_out0.at[pl.ds(row0, ROWS_PT)])

    @pl.when(c == 1)
    def _():
        pltpu.sync_copy(acc_sh.at[pl.ds(row0, ROWS_PT)],
                        acc_out1.at[pl.ds(row0, ROWS_PT)])


_scat_call = functools.partial(
    pl.kernel,
    out_type=[jax.ShapeDtypeStruct((NPAD, D), jnp.float32),
              jax.ShapeDtypeStruct((NPAD, D), jnp.float32)],
    mesh=_mesh,
    scratch_types=[
        pltpu.VMEM((SUP, EC), jnp.int32),
        pltpu.VMEM((SUP, EC), jnp.int32),
        pltpu.VMEM((EC, D), jnp.float32),
        pltpu.VMEM((EC, D), jnp.float32),
        pltpu.VMEM_SHARED((NPAD, D), jnp.float32),
        pltpu.SemaphoreType.DMA,
        pltpu.SemaphoreType.DMA,
    ],
)(_scat_body)


def _tc1_body(x_ref, w_ref, d0_ref, d1_ref, g_ref, dinv_ref):
    deg = jnp.maximum(d0_ref[:, 0:1] + d1_ref[:, 0:1] + 1.0, 1.0)
    dinv = lax.rsqrt(deg)
    h = jnp.dot(x_ref[...], w_ref[...], preferred_element_type=jnp.float32)
    g_ref[...] = h * dinv
    dinv_ref[...] = jnp.broadcast_to(dinv, dinv_ref.shape)


def _tc1(x, w1, d0, d1):
    return pl.pallas_call(
        _tc1_body,
        grid=(GRID,),
        in_specs=[
            pl.BlockSpec((RB, D), lambda i: (i, 0)),
            pl.BlockSpec((D, D), lambda i: (0, 0)),
            pl.BlockSpec((RB, D), lambda i: (i, 0)),
            pl.BlockSpec((RB, D), lambda i: (i, 0)),
        ],
        out_specs=[
            pl.BlockSpec((RB, D), lambda i: (i, 0)),
            pl.BlockSpec((RB, 16), lambda i: (i, 0)),
        ],
        out_shape=[
            jax.ShapeDtypeStruct((NPAD, D), jnp.float32),
            jax.ShapeDtypeStruct((NPAD, 16), jnp.float32),
        ],
    )(x, w1, d0, d1)


def _tc2_body(a0_ref, a1_ref, g_ref, dinv_ref, b_ref, w_ref, g2_ref):
    i = pl.program_id(0)
    dinv = dinv_ref[:, 0:1]
    z = dinv * (a0_ref[...] + a1_ref[...] + g_ref[...]) + b_ref[...]
    z = jnp.maximum(z, 0.0)
    rows = i * RB + lax.broadcasted_iota(jnp.int32, (RB, 1), 0)
    z = jnp.where(rows < N, z, 0.0)
    h2 = jnp.dot(z, w_ref[...], preferred_element_type=jnp.float32)
    g2_ref[...] = h2 * dinv


def _tc3_n_body(a0_ref, a1_ref, g_ref, dinv_ref, b_ref, out_ref):
    dinv = dinv_ref[:, 0:1]
    z = dinv * (a0_ref[...] + a1_ref[...] + g_ref[...]) + b_ref[...]
    out_ref[...] = jnp.maximum(z, 0.0)


def _tc3_n(a0, a1, g, dinv, b2):
    nb = 2000
    return pl.pallas_call(
        _tc3_n_body,
        grid=(N // nb,),
        in_specs=[
            pl.BlockSpec((nb, D), lambda i: (i, 0)),
            pl.BlockSpec((nb, D), lambda i: (i, 0)),
            pl.BlockSpec((nb, D), lambda i: (i, 0)),
            pl.BlockSpec((nb, 16), lambda i: (i, 0)),
            pl.BlockSpec((1, D), lambda i: (0, 0)),
        ],
        out_specs=pl.BlockSpec((nb, D), lambda i: (i, 0)),
        out_shape=jax.ShapeDtypeStruct((N, D), jnp.float32),
    )(a0, a1, g, dinv, b2)


def _tc2(a0, a1, g, dinv, b1, w2):
    return pl.pallas_call(
        _tc2_body,
        grid=(GRID,),
        in_specs=[
            pl.BlockSpec((RB, D), lambda i: (i, 0)),
            pl.BlockSpec((RB, D), lambda i: (i, 0)),
            pl.BlockSpec((RB, D), lambda i: (i, 0)),
            pl.BlockSpec((RB, 16), lambda i: (i, 0)),
            pl.BlockSpec((1, D), lambda i: (0, 0)),
            pl.BlockSpec((D, D), lambda i: (0, 0)),
        ],
        out_specs=pl.BlockSpec((RB, D), lambda i: (i, 0)),
        out_shape=jax.ShapeDtypeStruct((NPAD, D), jnp.float32),
    )(a0, a1, g, dinv, b1, w2)


def kernel(x, edge_index, W1, b1, W2, b2):
    src = edge_index[0].astype(jnp.int32)
    dst = edge_index[1].astype(jnp.int32)
    src_p = src.reshape(NW * NSUP, SUP, EC)
    dst_p = dst.reshape(NW * NSUP, SUP, EC)

    x_p = jnp.pad(x, ((0, NPAD - N), (0, 0)))
    b1r = b1.reshape(1, D)
    b2r = b2.reshape(1, D)

    d0, d1 = _deg_call(dst_p)
    g1, dinv = _tc1(x_p, W1, d0, d1)
    a0, a1 = _scat_call(g1, src_p, dst_p)
    g2 = _tc2(a0, a1, g1, dinv, b1r, W2)
    a0, a1 = _scat_call(g2, src_p, dst_p)
    return _tc3_n(a0, a1, g2, dinv, b2r)
